# baseline (speedup 1.0000x reference)
import functools

import jax
import jax.numpy as jnp
from jax import lax
from jax.experimental import pallas as pl
from jax.experimental.pallas import tpu as pltpu

_B, _N, _M = 4, 1024, 1024
_SROW = _M + 1
_GAMMA = 0.5
_LAMDA = 0.5

_RB = 256
_NRB = 5
_CB = 256
_NCB = 4
_NSTEP = _NRB + _NCB
_KCNT = float(2 * _B * _N * (_M - 1))


def _i32(v):
    return jnp.int32(v)


def _rowsum(a):
    ones = jnp.ones((a.shape[1], 1), jnp.float32)
    return jax.lax.dot_general(a, ones, (((1,), (0,)), ((), ())),
                               preferred_element_type=jnp.float32)


def _colsum(a):
    ones = jnp.ones((1, a.shape[0]), jnp.float32)
    return jax.lax.dot_general(ones, a, (((1,), (0,)), ((), ())),
                               preferred_element_type=jnp.float32)


def _pos01(x):
    return (x > 0.0).astype(jnp.float32)


def _body(srow_ref, d_ref, scol_ref, gt0_ref, gt1_ref, gt1c_ref, out_ref,
          colS, colQ, colP, acc):
    b = pl.program_id(0)
    step = pl.program_id(1)

    @pl.when(jnp.logical_and(b == 0, step == 0))
    def _init_acc():
        acc[0] = 0.0
        acc[1] = 0.0
        acc[2] = 0.0
        acc[3] = 0.0
        acc[4] = 0.0

    @pl.when(step == 0)
    def _init_cols():
        z = jnp.zeros((1, _M), jnp.float32)
        colS[...] = z
        colQ[...] = z
        colP[...] = z

    @pl.when(step < _NRB)
    def _phase_a():
        S = srow_ref[0]
        gt0 = gt0_ref[0]
        rowid = step * _RB + lax.broadcasted_iota(jnp.int32, (_RB, 1), 0)
        cid = lax.broadcasted_iota(jnp.int32, (_RB, _SROW), 1)
        onehot = cid == gt0
        s_pos0 = _rowsum(jnp.where(onehot, S, 0.0))
        x = S - (s_pos0 - _GAMMA)
        T0 = _rowsum(jnp.maximum(x, 0.0))
        C0 = _rowsum(_pos01(x))
        rowterm = (T0 - _GAMMA) / jnp.maximum(C0 - 1.0, 1.0)
        acc[0] += jnp.sum(jnp.where(rowid < _N, rowterm, 0.0))

        @pl.when(step < _NRB - 1)
        def _dist_rows():
            D = d_ref[0]
            D2 = D * D
            oh = onehot[:, :_M]
            d_pos0 = _rowsum(jnp.where(oh, D, 0.0))
            RS = _rowsum(D)
            RQ = _rowsum(D2)
            acc[2] += jnp.sum(float(_M) * d_pos0 - RS)
            acc[3] += jnp.sum(float(_M) * d_pos0 * d_pos0
                              - 2.0 * d_pos0 * RS + RQ)
            gt1 = gt1_ref[0]
            oh1 = rowid == gt1
            colS[...] += _colsum(D)
            colQ[...] += _colsum(D2)
            colP[...] += _colsum(jnp.where(oh1, D, 0.0))

        @pl.when(step == _NRB - 1)
        def _fin_dist_cols():
            CS = colS[...]
            CQ = colQ[...]
            d_pos1 = colP[...]
            acc[2] += jnp.sum(float(_N) * d_pos1 - CS)
            acc[3] += jnp.sum(float(_N) * d_pos1 * d_pos1
                              - 2.0 * d_pos1 * CS + CQ)

    @pl.when(step >= _NRB)
    def _phase_b():
        Sc = scol_ref[0]
        gt1c = gt1c_ref[0]
        rid_s = lax.broadcasted_iota(jnp.int32, (_SROW, _CB), 0)
        onehot1 = rid_s == gt1c
        s_pos1 = _colsum(jnp.where(onehot1, Sc, 0.0))
        y = Sc - (s_pos1 - _GAMMA)
        T1 = _colsum(jnp.maximum(y, 0.0))
        C1 = _colsum(_pos01(y))
        colterm = (T1 - _GAMMA) / jnp.maximum(C1 - 1.0, 1.0)
        acc[1] += jnp.sum(colterm)
        acc[4] += jnp.sum(s_pos1)

    @pl.when(jnp.logical_and(b == _B - 1, step == _NSTEP - 1))
    def _final():
        denom = float(_B * _N)
        gap_total = (acc[0] / denom + acc[1] / denom) * 0.5
        ot_loss = -acc[4] / denom
        mean_margin = acc[2] / _KCNT
        var_loss = (acc[3] - acc[2] * acc[2] / _KCNT) / (_KCNT - 1.0)
        aml = jnp.exp(mean_margin)
        loss = ((ot_loss + aml + var_loss) * (1.0 - _LAMDA)
                + (gap_total + var_loss) * _LAMDA)
        out_ref[...] = jnp.reshape(loss, (1, 1))


def _clamp_col(step):
    return jnp.clip(step - _NRB, 0, _NCB - 1).astype(jnp.int32)


_call_kwargs = dict(
    grid=(_B, _NSTEP),
    in_specs=[
        pl.BlockSpec((1, _RB, _SROW),
                     lambda b, s: (b, jnp.minimum(s, _NRB - 1).astype(jnp.int32),
                                   _i32(0))),
        pl.BlockSpec((1, _RB, _M),
                     lambda b, s: (b, jnp.minimum(s, _NRB - 2).astype(jnp.int32),
                                   _i32(0))),
        pl.BlockSpec((1, _SROW, _CB), lambda b, s: (b, _i32(0), _clamp_col(s))),
        pl.BlockSpec((1, _RB, 1),
                     lambda b, s: (b, jnp.minimum(s, _NRB - 1).astype(jnp.int32),
                                   _i32(0))),
        pl.BlockSpec((1, 1, _M), lambda b, s: (b, _i32(0), _i32(0))),
        pl.BlockSpec((1, 1, _CB), lambda b, s: (b, _i32(0), _clamp_col(s))),
    ],
    out_specs=pl.BlockSpec((1, 1), lambda b, s: (_i32(0), _i32(0))),
    out_shape=jax.ShapeDtypeStruct((1, 1), jnp.float32),
    scratch_shapes=[
        pltpu.VMEM((1, _M), jnp.float32),
        pltpu.VMEM((1, _M), jnp.float32),
        pltpu.VMEM((1, _M), jnp.float32),
        pltpu.SMEM((8,), jnp.float32),
    ],
    compiler_params=pltpu.CompilerParams(
        dimension_semantics=("arbitrary", "arbitrary")),
)


@functools.cache
def _make_call():
    return pl.pallas_call(_body, **_call_kwargs)


def kernel(gt_matches0, gt_matches1, scores, distance):
    scores = scores.astype(jnp.float32)
    distance = distance.astype(jnp.float32)
    pad = _NRB * _RB - _N
    gt0 = jnp.pad(gt_matches0.astype(jnp.int32), ((0, 0), (0, pad)),
                  constant_values=-1)[..., None]
    gt1 = gt_matches1.astype(jnp.int32)[:, None, :]

    out = _make_call()(scores, distance, scores, gt0, gt1, gt1)
    return out[0, 0]


# speedup vs baseline: 1.3250x; 1.3250x over previous
import functools

import jax
import jax.numpy as jnp
from jax import lax
from jax.experimental import pallas as pl
from jax.experimental.pallas import tpu as pltpu

_B, _N, _M = 4, 1024, 1024
_SROW = _M + 1
_GAMMA = 0.5
_LAMDA = 0.5

_RB = 256
_NRB = 5
_CB = 512
_NCB = 2
_NSTEP = _NRB + _NCB
_KCNT = float(2 * _B * _N * (_M - 1))


def _i32(v):
    return jnp.int32(v)


def _rowsum(a):
    return jnp.sum(a, axis=1, keepdims=True)


def _colsum(a):
    return jnp.sum(a, axis=0, keepdims=True)


def _pos01(x):
    return (x > 0.0).astype(jnp.float32)


def _body(srow_ref, d_ref, scol_ref, gt0_ref, gt1_ref, gt1c_ref, out_ref,
          colS, colQ, colP, acc):
    b = pl.program_id(0)
    step = pl.program_id(1)

    @pl.when(jnp.logical_and(b == 0, step == 0))
    def _init_acc():
        acc[0] = 0.0
        acc[1] = 0.0
        acc[2] = 0.0
        acc[3] = 0.0
        acc[4] = 0.0

    @pl.when(step == 0)
    def _init_cols():
        z = jnp.zeros((1, _M), jnp.float32)
        colS[...] = z
        colQ[...] = z
        colP[...] = z

    @pl.when(step < _NRB)
    def _phase_a():
        S = srow_ref[0]
        gt0 = gt0_ref[0]
        rowid = step * _RB + lax.broadcasted_iota(jnp.int32, (_RB, 1), 0)
        cid = lax.broadcasted_iota(jnp.int32, (_RB, _SROW), 1)
        onehot = cid == gt0
        s_pos0 = _rowsum(jnp.where(onehot, S, 0.0))
        x = S - (s_pos0 - _GAMMA)
        T0 = _rowsum(jnp.maximum(x, 0.0))
        C0 = _rowsum(_pos01(x))
        rowterm = (T0 - _GAMMA) / jnp.maximum(C0 - 1.0, 1.0)
        acc[0] += jnp.sum(jnp.where(rowid < _N, rowterm, 0.0))

        @pl.when(step < _NRB - 1)
        def _dist_rows():
            D = d_ref[0]
            D2 = D * D
            oh = onehot[:, :_M]
            d_pos0 = _rowsum(jnp.where(oh, D, 0.0))
            RS = _rowsum(D)
            RQ = _rowsum(D2)
            acc[2] += jnp.sum(float(_M) * d_pos0 - RS)
            acc[3] += jnp.sum(float(_M) * d_pos0 * d_pos0
                              - 2.0 * d_pos0 * RS + RQ)
            gt1 = gt1_ref[0]
            oh1 = rowid == gt1
            colS[...] += _colsum(D)
            colQ[...] += _colsum(D2)
            colP[...] += _colsum(jnp.where(oh1, D, 0.0))

        @pl.when(step == _NRB - 1)
        def _fin_dist_cols():
            CS = colS[...]
            CQ = colQ[...]
            d_pos1 = colP[...]
            acc[2] += jnp.sum(float(_N) * d_pos1 - CS)
            acc[3] += jnp.sum(float(_N) * d_pos1 * d_pos1
                              - 2.0 * d_pos1 * CS + CQ)

    @pl.when(step >= _NRB)
    def _phase_b():
        Sc = scol_ref[0]
        gt1c = gt1c_ref[0]
        rid_s = lax.broadcasted_iota(jnp.int32, (_SROW, _CB), 0)
        onehot1 = rid_s == gt1c
        s_pos1 = _colsum(jnp.where(onehot1, Sc, 0.0))
        y = Sc - (s_pos1 - _GAMMA)
        T1 = _colsum(jnp.maximum(y, 0.0))
        C1 = _colsum(_pos01(y))
        colterm = (T1 - _GAMMA) / jnp.maximum(C1 - 1.0, 1.0)
        acc[1] += jnp.sum(colterm)
        acc[4] += jnp.sum(s_pos1)

    @pl.when(jnp.logical_and(b == _B - 1, step == _NSTEP - 1))
    def _final():
        denom = float(_B * _N)
        gap_total = (acc[0] / denom + acc[1] / denom) * 0.5
        ot_loss = -acc[4] / denom
        mean_margin = acc[2] / _KCNT
        var_loss = (acc[3] - acc[2] * acc[2] / _KCNT) / (_KCNT - 1.0)
        aml = jnp.exp(mean_margin)
        loss = ((ot_loss + aml + var_loss) * (1.0 - _LAMDA)
                + (gap_total + var_loss) * _LAMDA)
        out_ref[...] = jnp.reshape(loss, (1, 1))


def _clamp_col(step):
    return jnp.clip(step - _NRB, 0, _NCB - 1).astype(jnp.int32)


_call_kwargs = dict(
    grid=(_B, _NSTEP),
    in_specs=[
        pl.BlockSpec((1, _RB, _SROW),
                     lambda b, s: (b, jnp.minimum(s, _NRB - 1).astype(jnp.int32),
                                   _i32(0))),
        pl.BlockSpec((1, _RB, _M),
                     lambda b, s: (b, jnp.minimum(s, _NRB - 2).astype(jnp.int32),
                                   _i32(0))),
        pl.BlockSpec((1, _SROW, _CB), lambda b, s: (b, _i32(0), _clamp_col(s))),
        pl.BlockSpec((1, _RB, 1),
                     lambda b, s: (b, jnp.minimum(s, _NRB - 1).astype(jnp.int32),
                                   _i32(0))),
        pl.BlockSpec((1, 1, _M), lambda b, s: (b, _i32(0), _i32(0))),
        pl.BlockSpec((1, 1, _CB), lambda b, s: (b, _i32(0), _clamp_col(s))),
    ],
    out_specs=pl.BlockSpec((1, 1), lambda b, s: (_i32(0), _i32(0))),
    out_shape=jax.ShapeDtypeStruct((1, 1), jnp.float32),
    scratch_shapes=[
        pltpu.VMEM((1, _M), jnp.float32),
        pltpu.VMEM((1, _M), jnp.float32),
        pltpu.VMEM((1, _M), jnp.float32),
        pltpu.SMEM((8,), jnp.float32),
    ],
    compiler_params=pltpu.CompilerParams(
        dimension_semantics=("arbitrary", "arbitrary")),
)


@functools.cache
def _make_call():
    return pl.pallas_call(_body, **_call_kwargs)


def kernel(gt_matches0, gt_matches1, scores, distance):
    scores = scores.astype(jnp.float32)
    distance = distance.astype(jnp.float32)
    pad = _NRB * _RB - _N
    gt0 = jnp.pad(gt_matches0.astype(jnp.int32), ((0, 0), (0, pad)),
                  constant_values=-1)[..., None]
    gt1 = gt_matches1.astype(jnp.int32)[:, None, :]

    out = _make_call()(scores, distance, scores, gt0, gt1, gt1)
    return out[0, 0]


# speedup vs baseline: 1.3828x; 1.0437x over previous
import functools

import jax
import jax.numpy as jnp
from jax import lax
from jax.experimental import pallas as pl
from jax.experimental.pallas import tpu as pltpu

_B, _N, _M = 4, 1024, 1024
_SROW = _M + 1
_GAMMA = 0.5
_LAMDA = 0.5

_RB = 344
_NRB = 3
_CB = 512
_NCB = 2
_NSTEP = _NRB + _NCB
_KCNT = float(2 * _B * _N * (_M - 1))


def _i32(v):
    return jnp.int32(v)


def _rowsum(a):
    return jnp.sum(a, axis=1, keepdims=True)


def _colsum(a):
    return jnp.sum(a, axis=0, keepdims=True)


def _pos01(x):
    return (x > 0.0).astype(jnp.float32)


def _body(srow_ref, d_ref, scol_ref, gt0_ref, gt1_ref, gt1c_ref, out_ref,
          colS, colQ, colP, acc):
    b = pl.program_id(0)
    step = pl.program_id(1)

    @pl.when(jnp.logical_and(b == 0, step == 0))
    def _init_acc():
        acc[0] = 0.0
        acc[1] = 0.0
        acc[2] = 0.0
        acc[3] = 0.0
        acc[4] = 0.0

    @pl.when(step == 0)
    def _init_cols():
        z = jnp.zeros((1, _M), jnp.float32)
        colS[...] = z
        colQ[...] = z
        colP[...] = z

    @pl.when(step < _NRB)
    def _phase_a():
        S = srow_ref[0]
        gt0 = gt0_ref[0]
        rowid = step * _RB + lax.broadcasted_iota(jnp.int32, (_RB, 1), 0)
        cid = lax.broadcasted_iota(jnp.int32, (_RB, _SROW), 1)
        onehot = cid == gt0
        s_pos0 = _rowsum(jnp.where(onehot, S, 0.0))
        x = S - (s_pos0 - _GAMMA)
        relu0 = jnp.maximum(x, 0.0)
        T0 = _rowsum(relu0)
        C0 = _rowsum(jnp.sign(relu0))
        rowterm = (T0 - _GAMMA) / jnp.maximum(C0 - 1.0, 1.0)
        acc[0] += jnp.sum(jnp.where(rowid < _N, rowterm, 0.0))

        def _dist_rows():
            D = jnp.where(rowid < _N, d_ref[0], 0.0)
            D2 = D * D
            oh = onehot[:, :_M]
            d_pos0 = _rowsum(jnp.where(oh, D, 0.0))
            RS = _rowsum(D)
            RQ = _rowsum(D2)
            acc[2] += jnp.sum(float(_M) * d_pos0 - RS)
            acc[3] += jnp.sum(float(_M) * d_pos0 * d_pos0
                              - 2.0 * d_pos0 * RS + RQ)
            gt1 = gt1_ref[0]
            oh1 = rowid == gt1
            colS[...] += _colsum(D)
            colQ[...] += _colsum(D2)
            colP[...] += _colsum(jnp.where(oh1, D, 0.0))
        _dist_rows()

        @pl.when(step == _NRB - 1)
        def _fin_dist_cols():
            CS = colS[...]
            CQ = colQ[...]
            d_pos1 = colP[...]
            acc[2] += jnp.sum(float(_N) * d_pos1 - CS)
            acc[3] += jnp.sum(float(_N) * d_pos1 * d_pos1
                              - 2.0 * d_pos1 * CS + CQ)

    @pl.when(step >= _NRB)
    def _phase_b():
        Sc = scol_ref[0]
        gt1c = gt1c_ref[0]
        rid_s = lax.broadcasted_iota(jnp.int32, (_SROW, _CB), 0)
        onehot1 = rid_s == gt1c
        s_pos1 = _colsum(jnp.where(onehot1, Sc, 0.0))
        y = Sc - (s_pos1 - _GAMMA)
        relu1 = jnp.maximum(y, 0.0)
        T1 = _colsum(relu1)
        C1 = _colsum(jnp.sign(relu1))
        colterm = (T1 - _GAMMA) / jnp.maximum(C1 - 1.0, 1.0)
        acc[1] += jnp.sum(colterm)
        acc[4] += jnp.sum(s_pos1)

    @pl.when(jnp.logical_and(b == _B - 1, step == _NSTEP - 1))
    def _final():
        denom = float(_B * _N)
        gap_total = (acc[0] / denom + acc[1] / denom) * 0.5
        ot_loss = -acc[4] / denom
        mean_margin = acc[2] / _KCNT
        var_loss = (acc[3] - acc[2] * acc[2] / _KCNT) / (_KCNT - 1.0)
        aml = jnp.exp(mean_margin)
        loss = ((ot_loss + aml + var_loss) * (1.0 - _LAMDA)
                + (gap_total + var_loss) * _LAMDA)
        out_ref[...] = jnp.reshape(loss, (1, 1))


def _clamp_col(step):
    return jnp.clip(step - _NRB, 0, _NCB - 1).astype(jnp.int32)


_call_kwargs = dict(
    grid=(_B, _NSTEP),
    in_specs=[
        pl.BlockSpec((1, _RB, _SROW),
                     lambda b, s: (b, jnp.minimum(s, _NRB - 1).astype(jnp.int32),
                                   _i32(0))),
        pl.BlockSpec((1, _RB, _M),
                     lambda b, s: (b, jnp.minimum(s, _NRB - 1).astype(jnp.int32),
                                   _i32(0))),
        pl.BlockSpec((1, _SROW, _CB), lambda b, s: (b, _i32(0), _clamp_col(s))),
        pl.BlockSpec((1, _RB, 1),
                     lambda b, s: (b, jnp.minimum(s, _NRB - 1).astype(jnp.int32),
                                   _i32(0))),
        pl.BlockSpec((1, 1, _M), lambda b, s: (b, _i32(0), _i32(0))),
        pl.BlockSpec((1, 1, _CB), lambda b, s: (b, _i32(0), _clamp_col(s))),
    ],
    out_specs=pl.BlockSpec((1, 1), lambda b, s: (_i32(0), _i32(0))),
    out_shape=jax.ShapeDtypeStruct((1, 1), jnp.float32),
    scratch_shapes=[
        pltpu.VMEM((1, _M), jnp.float32),
        pltpu.VMEM((1, _M), jnp.float32),
        pltpu.VMEM((1, _M), jnp.float32),
        pltpu.SMEM((8,), jnp.float32),
    ],
    compiler_params=pltpu.CompilerParams(
        dimension_semantics=("arbitrary", "arbitrary")),
)


@functools.cache
def _make_call():
    return pl.pallas_call(_body, **_call_kwargs)


def kernel(gt_matches0, gt_matches1, scores, distance):
    scores = scores.astype(jnp.float32)
    distance = distance.astype(jnp.float32)
    pad = _NRB * _RB - _N
    gt0 = jnp.pad(gt_matches0.astype(jnp.int32), ((0, 0), (0, pad)),
                  constant_values=-1)[..., None]
    gt1 = gt_matches1.astype(jnp.int32)[:, None, :]

    out = _make_call()(scores, distance, scores, gt0, gt1, gt1)
    return out[0, 0]


# speedup vs baseline: 1.3871x; 1.0031x over previous
import functools

import jax
import jax.numpy as jnp
from jax import lax
from jax.experimental import pallas as pl
from jax.experimental.pallas import tpu as pltpu

_B, _N, _M = 4, 1024, 1024
_SROW = _M + 1
_GAMMA = 0.5
_LAMDA = 0.5

_RB = 344
_NRB = 3
_CB = 512
_NCB = 2
_NSTEP = _NRB + _NCB
_KCNT = float(2 * _B * _N * (_M - 1))


def _i32(v):
    return jnp.int32(v)


def _rowsum(a):
    return jnp.sum(a, axis=1, keepdims=True)


def _colsum(a):
    return jnp.sum(a, axis=0, keepdims=True)


def _pos01(x):
    return (x > 0.0).astype(jnp.float32)


def _body(srow_ref, d_ref, scol_ref, gt0_ref, gt1_ref, gt1c_ref, out_ref,
          colS, colQ, colP, colSP, acc):
    b = pl.program_id(0)
    step = pl.program_id(1)

    @pl.when(jnp.logical_and(b == 0, step == 0))
    def _init_acc():
        acc[0] = 0.0
        acc[1] = 0.0
        acc[2] = 0.0
        acc[3] = 0.0
        acc[4] = 0.0

    @pl.when(step == 0)
    def _init_cols():
        z = jnp.zeros((1, _M), jnp.float32)
        colS[...] = z
        colQ[...] = z
        colP[...] = z
        colSP[...] = z

    @pl.when(step < _NRB)
    def _phase_a():
        S = srow_ref[0]
        gt0 = gt0_ref[0]
        rowid = step * _RB + lax.broadcasted_iota(jnp.int32, (_RB, 1), 0)
        cid = lax.broadcasted_iota(jnp.int32, (_RB, _SROW), 1)
        onehot = cid == gt0
        s_pos0 = _rowsum(jnp.where(onehot, S, 0.0))
        x = S - (s_pos0 - _GAMMA)
        relu0 = jnp.maximum(x, 0.0)
        T0 = _rowsum(relu0)
        C0 = _rowsum(jnp.sign(relu0))
        rowterm = (T0 - _GAMMA) / jnp.maximum(C0 - 1.0, 1.0)
        acc[0] += jnp.sum(jnp.where(rowid < _N, rowterm, 0.0))

        def _dist_rows():
            D = jnp.where(rowid < _N, d_ref[0], 0.0)
            D2 = D * D
            oh = onehot[:, :_M]
            d_pos0 = _rowsum(jnp.where(oh, D, 0.0))
            RS = _rowsum(D)
            RQ = _rowsum(D2)
            acc[2] += jnp.sum(float(_M) * d_pos0 - RS)
            acc[3] += jnp.sum(float(_M) * d_pos0 * d_pos0
                              - 2.0 * d_pos0 * RS + RQ)
            gt1 = gt1_ref[0]
            oh1 = rowid == gt1
            colS[...] += _colsum(D)
            colQ[...] += _colsum(D2)
            colP[...] += _colsum(jnp.where(oh1, D, 0.0))
            colSP[...] += _colsum(jnp.where(oh1, S[:, :_M], 0.0))
        _dist_rows()

        @pl.when(step == _NRB - 1)
        def _fin_dist_cols():
            CS = colS[...]
            CQ = colQ[...]
            d_pos1 = colP[...]
            acc[2] += jnp.sum(float(_N) * d_pos1 - CS)
            acc[3] += jnp.sum(float(_N) * d_pos1 * d_pos1
                              - 2.0 * d_pos1 * CS + CQ)
            acc[4] += jnp.sum(colSP[...])

    @pl.when(step >= _NRB)
    def _phase_b():
        Sc = scol_ref[0]
        s_pos1 = colSP[:, pl.ds((step - _NRB) * _CB, _CB)]
        y = Sc - (s_pos1 - _GAMMA)
        relu1 = jnp.maximum(y, 0.0)
        T1 = _colsum(relu1)
        C1 = _colsum(jnp.sign(relu1))
        colterm = (T1 - _GAMMA) / jnp.maximum(C1 - 1.0, 1.0)
        acc[1] += jnp.sum(colterm)

    @pl.when(jnp.logical_and(b == _B - 1, step == _NSTEP - 1))
    def _final():
        denom = float(_B * _N)
        gap_total = (acc[0] / denom + acc[1] / denom) * 0.5
        ot_loss = -acc[4] / denom
        mean_margin = acc[2] / _KCNT
        var_loss = (acc[3] - acc[2] * acc[2] / _KCNT) / (_KCNT - 1.0)
        aml = jnp.exp(mean_margin)
        loss = ((ot_loss + aml + var_loss) * (1.0 - _LAMDA)
                + (gap_total + var_loss) * _LAMDA)
        out_ref[...] = jnp.reshape(loss, (1, 1))


def _clamp_col(step):
    return jnp.clip(step - _NRB, 0, _NCB - 1).astype(jnp.int32)


_call_kwargs = dict(
    grid=(_B, _NSTEP),
    in_specs=[
        pl.BlockSpec((1, _RB, _SROW),
                     lambda b, s: (b, jnp.minimum(s, _NRB - 1).astype(jnp.int32),
                                   _i32(0))),
        pl.BlockSpec((1, _RB, _M),
                     lambda b, s: (b, jnp.minimum(s, _NRB - 1).astype(jnp.int32),
                                   _i32(0))),
        pl.BlockSpec((1, _SROW, _CB), lambda b, s: (b, _i32(0), _clamp_col(s))),
        pl.BlockSpec((1, _RB, 1),
                     lambda b, s: (b, jnp.minimum(s, _NRB - 1).astype(jnp.int32),
                                   _i32(0))),
        pl.BlockSpec((1, 1, _M), lambda b, s: (b, _i32(0), _i32(0))),
        pl.BlockSpec((1, 1, _CB), lambda b, s: (b, _i32(0), _clamp_col(s))),
    ],
    out_specs=pl.BlockSpec((1, 1), lambda b, s: (_i32(0), _i32(0))),
    out_shape=jax.ShapeDtypeStruct((1, 1), jnp.float32),
    scratch_shapes=[
        pltpu.VMEM((1, _M), jnp.float32),
        pltpu.VMEM((1, _M), jnp.float32),
        pltpu.VMEM((1, _M), jnp.float32),
        pltpu.VMEM((1, _M), jnp.float32),
        pltpu.SMEM((8,), jnp.float32),
    ],
    compiler_params=pltpu.CompilerParams(
        dimension_semantics=("arbitrary", "arbitrary")),
)


@functools.cache
def _make_call():
    return pl.pallas_call(_body, **_call_kwargs)


def kernel(gt_matches0, gt_matches1, scores, distance):
    scores = scores.astype(jnp.float32)
    distance = distance.astype(jnp.float32)
    pad = _NRB * _RB - _N
    gt0 = jnp.pad(gt_matches0.astype(jnp.int32), ((0, 0), (0, pad)),
                  constant_values=-1)[..., None]
    gt1 = gt_matches1.astype(jnp.int32)[:, None, :]

    out = _make_call()(scores, distance, scores, gt0, gt1, gt1)
    return out[0, 0]


# speedup vs baseline: 1.4766x; 1.0646x over previous
import functools

import jax
import jax.numpy as jnp
from jax import lax
from jax.experimental import pallas as pl
from jax.experimental.pallas import tpu as pltpu

_B, _N, _M = 4, 1024, 1024
_SROW = _M + 1
_GAMMA = 0.5
_LAMDA = 0.5

_RB = 344
_NRB = 3
_CB = 512
_NCB = 2
_NSTEP = _NRB + _NCB
_KCNT = float(2 * _B * _N * (_M - 1))


def _i32(v):
    return jnp.int32(v)


def _rowsum(a):
    return jnp.sum(a, axis=1, keepdims=True)


def _colsum(a):
    return jnp.sum(a, axis=0, keepdims=True)


def _pos01(x):
    return (x > 0.0).astype(jnp.float32)


def _body(srow_ref, d_ref, scol_ref, gt0_ref, gt1_ref, gt1c_ref, out_ref,
          colS, colQ, colP, colSP, acc):
    b = pl.program_id(0)
    step = pl.program_id(1)

    @pl.when(jnp.logical_and(b == 0, step == 0))
    def _init_acc():
        acc[0] = 0.0
        acc[1] = 0.0
        acc[2] = 0.0
        acc[3] = 0.0
        acc[4] = 0.0

    @pl.when(step == 0)
    def _init_cols():
        z = jnp.zeros((1, _M), jnp.float32)
        colS[...] = z
        colQ[...] = z
        colP[...] = z
        colSP[...] = z

    @pl.when(step < _NRB)
    def _phase_a():
        S = srow_ref[0]
        gt0 = gt0_ref[0]
        rowid = step * _RB + lax.broadcasted_iota(jnp.int32, (_RB, 1), 0)
        cid = lax.broadcasted_iota(jnp.int32, (_RB, _SROW), 1)
        onehot = cid == gt0
        s_pos0 = _rowsum(jnp.where(onehot, S, 0.0))
        x = S - (s_pos0 - _GAMMA)
        relu0 = jnp.maximum(x, 0.0)
        T0 = _rowsum(relu0)
        C0 = _rowsum(_pos01(x))
        rowterm = (T0 - _GAMMA) / jnp.maximum(C0 - 1.0, 1.0)
        acc[0] += jnp.sum(jnp.where(rowid < _N, rowterm, 0.0))

        def _dist_rows():
            D = jnp.where(rowid < _N, d_ref[0], 0.0)
            D2 = D * D
            oh = onehot[:, :_M]
            d_pos0 = _rowsum(jnp.where(oh, D, 0.0))
            RS = _rowsum(D)
            RQ = _rowsum(D2)
            acc[2] += jnp.sum(float(_M) * d_pos0 - RS)
            acc[3] += jnp.sum(float(_M) * d_pos0 * d_pos0
                              - 2.0 * d_pos0 * RS + RQ)
            gt1 = gt1_ref[0]
            oh1 = rowid == gt1
            colS[...] += _colsum(D)
            colQ[...] += _colsum(D2)
            colP[...] += _colsum(jnp.where(oh1, D, 0.0))
            colSP[...] += _colsum(jnp.where(oh1, S[:, :_M], 0.0))
        _dist_rows()

        @pl.when(step == _NRB - 1)
        def _fin_dist_cols():
            CS = colS[...]
            CQ = colQ[...]
            d_pos1 = colP[...]
            acc[2] += jnp.sum(float(_N) * d_pos1 - CS)
            acc[3] += jnp.sum(float(_N) * d_pos1 * d_pos1
                              - 2.0 * d_pos1 * CS + CQ)
            acc[4] += jnp.sum(colSP[...])

    @pl.when(step >= _NRB)
    def _phase_b():
        Sc = scol_ref[0]
        s_pos1 = colSP[:, pl.ds((step - _NRB) * _CB, _CB)]
        y = Sc - (s_pos1 - _GAMMA)
        relu1 = jnp.maximum(y, 0.0)
        T1 = _colsum(relu1)
        C1 = _colsum(_pos01(y))
        colterm = (T1 - _GAMMA) / jnp.maximum(C1 - 1.0, 1.0)
        acc[1] += jnp.sum(colterm)

    @pl.when(jnp.logical_and(b == _B - 1, step == _NSTEP - 1))
    def _final():
        denom = float(_B * _N)
        gap_total = (acc[0] / denom + acc[1] / denom) * 0.5
        ot_loss = -acc[4] / denom
        mean_margin = acc[2] / _KCNT
        var_loss = (acc[3] - acc[2] * acc[2] / _KCNT) / (_KCNT - 1.0)
        aml = jnp.exp(mean_margin)
        loss = ((ot_loss + aml + var_loss) * (1.0 - _LAMDA)
                + (gap_total + var_loss) * _LAMDA)
        out_ref[...] = jnp.reshape(loss, (1, 1))


def _clamp_col(step):
    return jnp.clip(step - _NRB, 0, _NCB - 1).astype(jnp.int32)


_call_kwargs = dict(
    grid=(_B, _NSTEP),
    in_specs=[
        pl.BlockSpec((1, _RB, _SROW),
                     lambda b, s: (b, jnp.minimum(s, _NRB - 1).astype(jnp.int32),
                                   _i32(0))),
        pl.BlockSpec((1, _RB, _M),
                     lambda b, s: (b, jnp.minimum(s, _NRB - 1).astype(jnp.int32),
                                   _i32(0))),
        pl.BlockSpec((1, _SROW, _CB), lambda b, s: (b, _i32(0), _clamp_col(s))),
        pl.BlockSpec((1, _RB, 1),
                     lambda b, s: (b, jnp.minimum(s, _NRB - 1).astype(jnp.int32),
                                   _i32(0))),
        pl.BlockSpec((1, 1, _M), lambda b, s: (b, _i32(0), _i32(0))),
        pl.BlockSpec((1, 1, _CB), lambda b, s: (b, _i32(0), _clamp_col(s))),
    ],
    out_specs=pl.BlockSpec((1, 1), lambda b, s: (_i32(0), _i32(0))),
    out_shape=jax.ShapeDtypeStruct((1, 1), jnp.float32),
    scratch_shapes=[
        pltpu.VMEM((1, _M), jnp.float32),
        pltpu.VMEM((1, _M), jnp.float32),
        pltpu.VMEM((1, _M), jnp.float32),
        pltpu.VMEM((1, _M), jnp.float32),
        pltpu.SMEM((8,), jnp.float32),
    ],
    compiler_params=pltpu.CompilerParams(
        dimension_semantics=("arbitrary", "arbitrary")),
)


@functools.cache
def _make_call():
    return pl.pallas_call(_body, **_call_kwargs)


def kernel(gt_matches0, gt_matches1, scores, distance):
    scores = scores.astype(jnp.float32)
    distance = distance.astype(jnp.float32)
    pad = _NRB * _RB - _N
    gt0 = jnp.pad(gt_matches0.astype(jnp.int32), ((0, 0), (0, pad)),
                  constant_values=-1)[..., None]
    gt1 = gt_matches1.astype(jnp.int32)[:, None, :]

    out = _make_call()(scores, distance, scores, gt0, gt1, gt1)
    return out[0, 0]
